# baseline (speedup 1.0000x reference)
import functools

import jax
import jax.numpy as jnp
from jax import lax
from jax.experimental import pallas as pl
from jax.experimental.pallas import tpu as pltpu
from jax.experimental.pallas import tpu_sc as plsc

_N = 10000
_E = 320000
_D = 128
_NP = 10240
_EP = 327680
_ER = _EP // 128
_CT = _ER // 16
_IC = 32
_IC2 = 16
_NT = _NP // 16 // 128
_RB = 1280
_GRID = _NP // _RB

_mesh = plsc.VectorSubcoreMesh(core_axis_name="c", subcore_axis_name="s")


@functools.partial(
    pl.kernel,
    mesh=_mesh,
    out_type=jax.ShapeDtypeStruct((2, _NP, 128), jnp.float32),
    scratch_types=[
        pltpu.VMEM((_IC, 128), jnp.int32),
        pltpu.VMEM((128, 128), jnp.float32),
        pltpu.VMEM((128, 128), jnp.float32),
        pltpu.VMEM_SHARED((_NP, 128), jnp.float32),
    ],
)
def _sc_degrees(eidx_hbm, ones_hbm, zeros_hbm, deg_hbm,
                sidx, ones_v, tmp, acc):
    core = lax.axis_index("c")
    sub = lax.axis_index("s")
    nrow0 = sub * (_NT * 128)
    erow0 = sub * _CT

    pltpu.sync_copy(zeros_hbm, tmp)
    for k in range(_NT):
        pltpu.sync_copy(tmp, acc.at[pl.ds(nrow0 + k * 128, 128)])
    pltpu.sync_copy(ones_hbm, ones_v)

    plsc.subcore_barrier()

    for m in range(_CT // _IC):
        pltpu.sync_copy(
            eidx_hbm.at[core].at[pl.ds(erow0 + m * _IC, _IC)], sidx)

        def body(j, carry):
            pltpu.sync_copy(ones_v, acc.at[sidx.at[j]], add=True)
            return carry

        lax.fori_loop(0, _IC, body, 0)

    plsc.subcore_barrier()

    for k in range(_NT):
        pltpu.sync_copy(acc.at[pl.ds(nrow0 + k * 128, 128)], tmp)
        pltpu.sync_copy(
            tmp, deg_hbm.at[core].at[pl.ds(nrow0 + k * 128, 128)])


@functools.partial(
    pl.kernel,
    mesh=_mesh,
    out_type=jax.ShapeDtypeStruct((2, _NP, 128), jnp.float32),
    scratch_types=[
        pltpu.VMEM((_IC2, 128), jnp.int32),
        pltpu.VMEM((_IC2, 128), jnp.int32),
        pltpu.VMEM((128, 128), jnp.float32),
        pltpu.VMEM((128, 128), jnp.float32),
        pltpu.VMEM_SHARED((_NP, 128), jnp.float32),
        pltpu.SemaphoreType.DMA,
        pltpu.SemaphoreType.DMA,
    ],
)
def _sc_segsum(s2_hbm, gidx_hbm, sidx_hbm, zeros_hbm, tx_hbm,
               ga, sc, buf_a, buf_b, acc, sem, sem_b):
    core = lax.axis_index("c")
    sub = lax.axis_index("s")
    nrow0 = sub * (_NT * 128)
    erow0 = sub * _CT

    pltpu.sync_copy(zeros_hbm, buf_a)
    for k in range(_NT):
        pltpu.sync_copy(buf_a, acc.at[pl.ds(nrow0 + k * 128, 128)])

    plsc.subcore_barrier()

    for m in range(_CT // _IC2):
        pltpu.sync_copy(
            gidx_hbm.at[core].at[pl.ds(erow0 + m * _IC2, _IC2)], ga)
        pltpu.sync_copy(
            sidx_hbm.at[core].at[pl.ds(erow0 + m * _IC2, _IC2)], sc)

        def body(t, carry):
            j = 2 * t
            pltpu.async_copy(s2_hbm.at[ga.at[j]], buf_a, sem)
            pltpu.async_copy(s2_hbm.at[ga.at[j + 1]], buf_b, sem_b)
            pltpu.make_async_copy(s2_hbm.at[ga.at[j]], buf_a, sem).wait()
            pltpu.sync_copy(buf_a, acc.at[sc.at[j]], add=True)
            pltpu.make_async_copy(
                s2_hbm.at[ga.at[j + 1]], buf_b, sem_b).wait()
            pltpu.sync_copy(buf_b, acc.at[sc.at[j + 1]], add=True)
            return carry
        lax.fori_loop(0, _IC2 // 2, body, 0)

    plsc.subcore_barrier()

    for k in range(_NT):
        pltpu.sync_copy(acc.at[pl.ds(nrow0 + k * 128, 128)], buf_a)
        pltpu.sync_copy(
            buf_a, tx_hbm.at[core].at[pl.ds(nrow0 + k * 128, 128)])


def _tc_scale_body(x_ref, deg_ref, s_ref):
    x = x_ref[...]
    s_ref[0] = x / deg_ref[0, :, 0:1]
    s_ref[1] = x / deg_ref[1, :, 0:1]


def _tc_scale(x_pad, deg):
    return pl.pallas_call(
        _tc_scale_body,
        grid=(_GRID,),
        in_specs=[
            pl.BlockSpec((_RB, 128), lambda i: (i, 0)),
            pl.BlockSpec((2, _RB, 128), lambda i: (0, i, 0)),
        ],
        out_specs=pl.BlockSpec((2, _RB, 128), lambda i: (0, i, 0)),
        out_shape=jax.ShapeDtypeStruct((2, _NP, 128), jnp.float32),
    )(x_pad, deg)


def _tc_gate_body(o_ref, tx_ref, wz_ref, wh_ref, bz_ref, bh_ref,
                  deg_ref, h_ref, s_ref):
    x3 = jnp.concatenate([o_ref[...], tx_ref[0], tx_ref[1]], axis=1)
    zlin = jnp.dot(x3, wz_ref[...], preferred_element_type=jnp.float32)
    hlin = jnp.dot(x3, wh_ref[...], preferred_element_type=jnp.float32)
    z = jax.nn.sigmoid(zlin + bz_ref[...])
    ht = jnp.tanh(hlin + bh_ref[...])
    h = (1.0 - z) * ht
    h_ref[...] = h
    s_ref[0] = h / deg_ref[0, :, 0:1]
    s_ref[1] = h / deg_ref[1, :, 0:1]


def _tc_gate(o, tx, wz, wh, bz, bh, deg):
    return pl.pallas_call(
        _tc_gate_body,
        grid=(_GRID,),
        in_specs=[
            pl.BlockSpec((_RB, 128), lambda i: (i, 0)),
            pl.BlockSpec((2, _RB, 128), lambda i: (0, i, 0)),
            pl.BlockSpec((384, 128), lambda i: (0, 0)),
            pl.BlockSpec((384, 128), lambda i: (0, 0)),
            pl.BlockSpec((1, 128), lambda i: (0, 0)),
            pl.BlockSpec((1, 128), lambda i: (0, 0)),
            pl.BlockSpec((2, _RB, 128), lambda i: (0, i, 0)),
        ],
        out_specs=[
            pl.BlockSpec((_RB, 128), lambda i: (i, 0)),
            pl.BlockSpec((2, _RB, 128), lambda i: (0, i, 0)),
        ],
        out_shape=[
            jax.ShapeDtypeStruct((_NP, 128), jnp.float32),
            jax.ShapeDtypeStruct((2, _NP, 128), jnp.float32),
        ],
    )(o, tx, wz, wh, bz, bh, deg)


def kernel(x, edge_index, Wz, Wr, Wh, bz, br, bh):
    del Wr, br
    src = edge_index[0]
    dst = edge_index[1]
    pad = jnp.full((_EP - _E,), _N, dtype=jnp.int32)
    src2d = jnp.concatenate([src, pad]).reshape(_ER, 128)
    dst2d = jnp.concatenate([dst, pad]).reshape(_ER, 128)
    eidx = jnp.stack([src2d, dst2d])
    gidx = jnp.stack([src2d, dst2d + _NP])
    sidx = jnp.stack([dst2d, src2d])
    x_pad = jnp.concatenate(
        [x, jnp.zeros((_NP - _N, _D), dtype=jnp.float32)], axis=0)
    ones128 = jnp.ones((128, 128), dtype=jnp.float32)
    zeros128 = jnp.zeros((128, 128), dtype=jnp.float32)

    def wcat(W, l):
        return jnp.concatenate(
            [W[l, 0, 0, :_D] + W[l, 1, 0, :_D], W[l, 0, 1, :_D],
             W[l, 1, 1, :_D]], axis=0)

    deg = _sc_degrees(eidx, ones128, zeros128)

    out = x_pad
    s2 = _tc_scale(x_pad, deg)
    hs = []
    for l in range(2):
        tx = _sc_segsum(s2.reshape(2 * _NP, 128), gidx, sidx, zeros128)
        out, s2 = _tc_gate(out, tx, wcat(Wz, l), wcat(Wh, l),
                           bz[l].reshape(1, 128), bh[l].reshape(1, 128),
                           deg)
        hs.append(out[:_N])
    return jnp.stack(hs)


# speedup vs baseline: 1.0772x; 1.0772x over previous
import functools

import jax
import jax.numpy as jnp
from jax import lax
from jax.experimental import pallas as pl
from jax.experimental.pallas import tpu as pltpu
from jax.experimental.pallas import tpu_sc as plsc

_N = 10000
_E = 320000
_D = 128
_NP = 10240
_EP = 327680
_ER = _EP // 128
_CT = _ER // 16
_IC = 32
_IC2 = 32
_NT = _NP // 16 // 128
_RB = 1280
_GRID = _NP // _RB

_mesh = plsc.VectorSubcoreMesh(core_axis_name="c", subcore_axis_name="s")


@functools.partial(
    pl.kernel,
    mesh=_mesh,
    out_type=jax.ShapeDtypeStruct((2, _NP, 128), jnp.float32),
    scratch_types=[
        pltpu.VMEM((_IC, 128), jnp.int32),
        pltpu.VMEM((128, 128), jnp.float32),
        pltpu.VMEM((128, 128), jnp.float32),
        pltpu.VMEM_SHARED((_NP, 128), jnp.float32),
    ],
)
def _sc_degrees(eidx_hbm, ones_hbm, zeros_hbm, deg_hbm,
                sidx, ones_v, tmp, acc):
    core = lax.axis_index("c")
    sub = lax.axis_index("s")
    nrow0 = sub * (_NT * 128)
    erow0 = sub * _CT

    pltpu.sync_copy(zeros_hbm, tmp)
    for k in range(_NT):
        pltpu.sync_copy(tmp, acc.at[pl.ds(nrow0 + k * 128, 128)])
    pltpu.sync_copy(ones_hbm, ones_v)

    plsc.subcore_barrier()

    for m in range(_CT // _IC):
        pltpu.sync_copy(
            eidx_hbm.at[core].at[pl.ds(erow0 + m * _IC, _IC)], sidx)

        def body(j, carry):
            pltpu.sync_copy(ones_v, acc.at[sidx.at[j]], add=True)
            return carry

        lax.fori_loop(0, _IC, body, 0)

    plsc.subcore_barrier()

    for k in range(_NT):
        pltpu.sync_copy(acc.at[pl.ds(nrow0 + k * 128, 128)], tmp)
        pltpu.sync_copy(
            tmp, deg_hbm.at[core].at[pl.ds(nrow0 + k * 128, 128)])


@functools.partial(
    pl.kernel,
    mesh=_mesh,
    out_type=jax.ShapeDtypeStruct((2, _NP, 128), jnp.float32),
    scratch_types=[
        pltpu.VMEM((_IC2, 128), jnp.int32),
        pltpu.VMEM((_IC2, 128), jnp.int32),
        pltpu.VMEM((128, 128), jnp.float32),
        pltpu.VMEM((128, 128), jnp.float32),
        pltpu.VMEM_SHARED((_NP, 128), jnp.float32),
        pltpu.SemaphoreType.DMA,
        pltpu.SemaphoreType.DMA,
    ],
)
def _sc_segsum(s2_hbm, gidx_hbm, sidx_hbm, zeros_hbm, tx_hbm,
               ga, sc, buf_a, buf_b, acc, sem, sem_b):
    core = lax.axis_index("c")
    sub = lax.axis_index("s")
    nrow0 = sub * (_NT * 128)
    erow0 = sub * _CT

    pltpu.sync_copy(zeros_hbm, buf_a)
    for k in range(_NT):
        pltpu.sync_copy(buf_a, acc.at[pl.ds(nrow0 + k * 128, 128)])

    plsc.subcore_barrier()

    for m in range(_CT // _IC2):
        pltpu.sync_copy(
            gidx_hbm.at[core].at[pl.ds(erow0 + m * _IC2, _IC2)], ga)
        pltpu.sync_copy(
            sidx_hbm.at[core].at[pl.ds(erow0 + m * _IC2, _IC2)], sc)

        pltpu.async_copy(s2_hbm.at[ga.at[0]], buf_a, sem)

        def body(t, carry):
            j = 2 * t
            pltpu.make_async_copy(s2_hbm.at[ga.at[j]], buf_a, sem).wait()
            pltpu.async_copy(s2_hbm.at[ga.at[j + 1]], buf_b, sem_b)
            pltpu.sync_copy(buf_a, acc.at[sc.at[j]], add=True)
            pltpu.make_async_copy(
                s2_hbm.at[ga.at[j + 1]], buf_b, sem_b).wait()
            pltpu.async_copy(s2_hbm.at[ga.at[j + 2]], buf_a, sem)
            pltpu.sync_copy(buf_b, acc.at[sc.at[j + 1]], add=True)
            return carry
        lax.fori_loop(0, _IC2 // 2 - 1, body, 0)

        jl = _IC2 - 2
        pltpu.make_async_copy(s2_hbm.at[ga.at[jl]], buf_a, sem).wait()
        pltpu.async_copy(s2_hbm.at[ga.at[jl + 1]], buf_b, sem_b)
        pltpu.sync_copy(buf_a, acc.at[sc.at[jl]], add=True)
        pltpu.make_async_copy(s2_hbm.at[ga.at[jl + 1]], buf_b, sem_b).wait()
        pltpu.sync_copy(buf_b, acc.at[sc.at[jl + 1]], add=True)

    plsc.subcore_barrier()

    for k in range(_NT):
        pltpu.sync_copy(acc.at[pl.ds(nrow0 + k * 128, 128)], buf_a)
        pltpu.sync_copy(
            buf_a, tx_hbm.at[core].at[pl.ds(nrow0 + k * 128, 128)])


def _tc_scale_body(x_ref, deg_ref, s_ref):
    x = x_ref[...]
    s_ref[0] = x / deg_ref[0, :, 0:1]
    s_ref[1] = x / deg_ref[1, :, 0:1]


def _tc_scale(x_pad, deg):
    return pl.pallas_call(
        _tc_scale_body,
        grid=(_GRID,),
        in_specs=[
            pl.BlockSpec((_RB, 128), lambda i: (i, 0)),
            pl.BlockSpec((2, _RB, 128), lambda i: (0, i, 0)),
        ],
        out_specs=pl.BlockSpec((2, _RB, 128), lambda i: (0, i, 0)),
        out_shape=jax.ShapeDtypeStruct((2, _NP, 128), jnp.float32),
    )(x_pad, deg)


def _tc_gate_body(o_ref, tx_ref, wz_ref, wh_ref, bz_ref, bh_ref,
                  deg_ref, h_ref, s_ref):
    x3 = jnp.concatenate([o_ref[...], tx_ref[0], tx_ref[1]], axis=1)
    zlin = jnp.dot(x3, wz_ref[...], preferred_element_type=jnp.float32)
    hlin = jnp.dot(x3, wh_ref[...], preferred_element_type=jnp.float32)
    z = jax.nn.sigmoid(zlin + bz_ref[...])
    ht = jnp.tanh(hlin + bh_ref[...])
    h = (1.0 - z) * ht
    h_ref[...] = h
    s_ref[0] = h / deg_ref[0, :, 0:1]
    s_ref[1] = h / deg_ref[1, :, 0:1]


def _tc_gate(o, tx, wz, wh, bz, bh, deg):
    return pl.pallas_call(
        _tc_gate_body,
        grid=(_GRID,),
        in_specs=[
            pl.BlockSpec((_RB, 128), lambda i: (i, 0)),
            pl.BlockSpec((2, _RB, 128), lambda i: (0, i, 0)),
            pl.BlockSpec((384, 128), lambda i: (0, 0)),
            pl.BlockSpec((384, 128), lambda i: (0, 0)),
            pl.BlockSpec((1, 128), lambda i: (0, 0)),
            pl.BlockSpec((1, 128), lambda i: (0, 0)),
            pl.BlockSpec((2, _RB, 128), lambda i: (0, i, 0)),
        ],
        out_specs=[
            pl.BlockSpec((_RB, 128), lambda i: (i, 0)),
            pl.BlockSpec((2, _RB, 128), lambda i: (0, i, 0)),
        ],
        out_shape=[
            jax.ShapeDtypeStruct((_NP, 128), jnp.float32),
            jax.ShapeDtypeStruct((2, _NP, 128), jnp.float32),
        ],
    )(o, tx, wz, wh, bz, bh, deg)


def kernel(x, edge_index, Wz, Wr, Wh, bz, br, bh):
    del Wr, br
    src = edge_index[0]
    dst = edge_index[1]
    pad = jnp.full((_EP - _E,), _N, dtype=jnp.int32)
    src2d = jnp.concatenate([src, pad]).reshape(_ER, 128)
    dst2d = jnp.concatenate([dst, pad]).reshape(_ER, 128)
    eidx = jnp.stack([src2d, dst2d])
    gidx = jnp.stack([src2d, dst2d + _NP])
    sidx = jnp.stack([dst2d, src2d])
    x_pad = jnp.concatenate(
        [x, jnp.zeros((_NP - _N, _D), dtype=jnp.float32)], axis=0)
    ones128 = jnp.ones((128, 128), dtype=jnp.float32)
    zeros128 = jnp.zeros((128, 128), dtype=jnp.float32)

    def wcat(W, l):
        return jnp.concatenate(
            [W[l, 0, 0, :_D] + W[l, 1, 0, :_D], W[l, 0, 1, :_D],
             W[l, 1, 1, :_D]], axis=0)

    deg = _sc_degrees(eidx, ones128, zeros128)

    out = x_pad
    s2 = _tc_scale(x_pad, deg)
    hs = []
    for l in range(2):
        tx = _sc_segsum(s2.reshape(2 * _NP, 128), gidx, sidx, zeros128)
        out, s2 = _tc_gate(out, tx, wcat(Wz, l), wcat(Wh, l),
                           bz[l].reshape(1, 128), bh[l].reshape(1, 128),
                           deg)
        hs.append(out[:_N])
    return jnp.stack(hs)
